# baseline (speedup 1.0000x reference)
import jax
import jax.numpy as jnp
from jax import lax
from jax.experimental import pallas as pl
from jax.experimental.pallas import tpu as pltpu
from jax.experimental.pallas import tpu_sc as plsc

VOCAB = 100000
PLANE_STRIDE = 100096
EMBED = 64
BATCH = 4096
SEQ = 50
NBT = BATCH // 128
NTILECOL = PLANE_STRIDE // 128

FLAT_TABLE = EMBED * PLANE_STRIDE
IDX_ROWS = SEQ * NBT


def _body(idx_hbm, table_hbm, out_hbm, tpl_hbm, iflat_hbm,
          inbuf, ibx, planebuf, sbuf,
          sem_in, sem_pw, sem_ib, sem_iw, sem_p, sem_i, sem_s):
    cid = lax.axis_index("c")
    sid = lax.axis_index("s")
    irbase = cid * IDX_ROWS

    for half in range(2):
        col = sid * 2 + half
        ilds = []
        for st in range(7):
            rows = 8 if st < 6 else 2
            ilds.append(pltpu.async_copy(
                idx_hbm.at[pl.ds(st * 8, rows),
                           pl.ds(pl.multiple_of(col * 128, 128), 128)],
                ibx.at[pl.ds(st * 8, rows)], sem_ib))
        for c in ilds:
            c.wait()

        def iw_fire(s, _):
            pltpu.async_copy(
                ibx.at[pl.ds(s, 1)],
                iflat_hbm.at[pl.ds(irbase + s * NBT + col, 1)], sem_iw)
            return 0

        def iw_drain(s, _):
            pltpu.make_async_copy(
                ibx.at[pl.ds(s, 1)],
                iflat_hbm.at[pl.ds(irbase + s * NBT + col, 1)],
                sem_iw).wait()
            return 0

        lax.fori_loop(0, SEQ, iw_fire, 0)
        lax.fori_loop(0, SEQ, iw_drain, 0)

    drow = cid * 32

    def plane_fire(c, p):
        def fire(dl, _):
            pltpu.async_copy(
                inbuf.at[p, dl],
                tpl_hbm.at[pl.ds((drow + dl) * PLANE_STRIDE + c * 128, 128)],
                sem_pw)
            return 0
        lax.fori_loop(0, 32, fire, 0)

    def plane_drain(c, p):
        def drain(dl, _):
            pltpu.make_async_copy(
                inbuf.at[p, dl],
                tpl_hbm.at[pl.ds((drow + dl) * PLANE_STRIDE + c * 128, 128)],
                sem_pw).wait()
            return 0
        lax.fori_loop(0, 32, drain, 0)

    def in_copy(k, p):
        c = k * 16 + sid
        return pltpu.async_copy(
            table_hbm.at[pl.ds(pl.multiple_of(drow, 8), 32),
                         pl.ds(pl.multiple_of(c * 128, 128), 128)],
            inbuf.at[p], sem_in)

    NCOL = 48
    loads = {0: in_copy(0, 0)}
    for k in range(NCOL):
        p = k % 2
        loads[k].wait()
        if k >= 1:
            plane_drain((k - 1) * 16 + sid, (k - 1) % 2)
        if k + 1 < NCOL:
            loads[k + 1] = in_copy(k + 1, (k + 1) % 2)
        plane_fire(k * 16 + sid, p)
    plane_drain((NCOL - 1) * 16 + sid, (NCOL - 1) % 2)

    @pl.when(sid < NTILECOL - NCOL * 16)
    def _tail():
        c = NCOL * 16 + sid
        pltpu.async_copy(
            table_hbm.at[pl.ds(pl.multiple_of(drow, 8), 32),
                         pl.ds(pl.multiple_of(c * 128, 128), 128)],
            inbuf.at[0], sem_in).wait()
        plane_fire(c, 0)
        plane_drain(c, 0)

    plsc.subcore_barrier()

    def do_plane(d):
        dt = lax.shift_right_logical(d, 3)
        d8 = lax.bitwise_and(d, 7)
        pltpu.async_copy(
            tpl_hbm.at[pl.ds(pl.multiple_of(d * PLANE_STRIDE, 8), VOCAB)],
            planebuf, sem_p).wait()

        def idx_copy(s, p):
            return (iflat_hbm.at[pl.ds(irbase + s * NBT, NBT)],
                    ibx.at[pl.ds(p * 32, 32)], sem_i)

        pltpu.async_copy(*idx_copy(0, 0))
        pltpu.async_copy(*idx_copy(1, 1))

        def qbody(q, _):
            for p in range(2):
                s = 2 * q + p
                pltpu.make_async_copy(*idx_copy(s, p)).wait()

                @pl.when(q > 0)
                def _wait_store():
                    pltpu.make_async_copy(
                        sbuf.at[p], out_hbm.at[s, dt, :, d8, :],
                        sem_s).wait()

                @plsc.parallel_loop(0, BATCH, step=16, unroll=8)
                def _gather(j):
                    bt = lax.shift_right_logical(j, 7)
                    off = lax.bitwise_and(j, 127)
                    ivals = ibx[p * 32 + bt, pl.ds(off, 16)]
                    vals = plsc.load_gather(planebuf, [ivals])
                    sbuf[p, bt, pl.ds(off, 16)] = vals

                pltpu.async_copy(
                    sbuf.at[p], out_hbm.at[s, dt, :, d8, :], sem_s)

                @pl.when(q < SEQ // 2 - 1)
                def _prefetch():
                    pltpu.async_copy(*idx_copy(s + 2, p))
            return 0

        lax.fori_loop(0, SEQ // 2, qbody, 0)
        for p in range(2):
            pltpu.make_async_copy(
                sbuf.at[p], out_hbm.at[0, dt, :, d8, :], sem_s).wait()

    do_plane(drow + sid * 2)
    do_plane(drow + sid * 2 + 1)


@jax.jit
def _lookup(idx_t, table_t):
    mesh = plsc.VectorSubcoreMesh(core_axis_name="c", subcore_axis_name="s")
    out5, _, _ = pl.kernel(
        _body,
        out_type=(
            jax.ShapeDtypeStruct((SEQ, 8, NBT, 8, 128), jnp.float32),
            jax.ShapeDtypeStruct((FLAT_TABLE,), jnp.float32),
            jax.ShapeDtypeStruct((2 * IDX_ROWS, 128), jnp.int32),
        ),
        mesh=mesh,
        scratch_types=[
            pltpu.VMEM((2, 32, 128), jnp.float32),
            pltpu.VMEM((64, 128), jnp.int32),
            pltpu.VMEM((VOCAB,), jnp.float32),
            pltpu.VMEM((2, NBT, 128), jnp.float32),
            pltpu.SemaphoreType.DMA,
            pltpu.SemaphoreType.DMA,
            pltpu.SemaphoreType.DMA,
            pltpu.SemaphoreType.DMA,
            pltpu.SemaphoreType.DMA,
            pltpu.SemaphoreType.DMA,
            pltpu.SemaphoreType.DMA,
        ],
        compiler_params=pltpu.CompilerParams(
            use_tc_tiling_on_sc=True, needs_layout_passes=False),
    )(idx_t, table_t)
    return out5


def kernel(token_type_ids, table):
    idx_t = jnp.transpose(token_type_ids, (1, 0))
    table_t = jnp.transpose(table, (1, 0))
    out5 = _lookup(idx_t, table_t)
    out = jnp.transpose(out5, (2, 4, 0, 1, 3))
    return jnp.reshape(out, (BATCH, SEQ, EMBED))


# speedup vs baseline: 1.1228x; 1.1228x over previous
import jax
import jax.numpy as jnp
from jax import lax
from jax.experimental import pallas as pl
from jax.experimental.pallas import tpu as pltpu
from jax.experimental.pallas import tpu_sc as plsc

VOCAB = 100000
PLANE_STRIDE = 100096
EMBED = 64
BATCH = 4096
SEQ = 50
NBT = BATCH // 128
NW = 32
NTILECOL = PLANE_STRIDE // 128

FLAT_TABLE = EMBED * PLANE_STRIDE
FLAT_IDX = SEQ * BATCH


def _detile_body(idx_hbm, table_hbm, tpl_hbm, iflat_hbm,
                 inbuf, ibuf, dummy, sem_in, sem_pw, sem_ib, sem_iw):
    w = lax.axis_index("s") * 2 + lax.axis_index("c")

    ilds = []
    for st in range(7):
        rows = 8 if st < 6 else 2
        ilds.append(pltpu.async_copy(
            idx_hbm.at[pl.ds(st * 8, rows),
                       pl.ds(pl.multiple_of(w * 128, 128), 128)],
            ibuf.at[pl.ds(st * 8, rows)], sem_ib))
    for c in ilds:
        c.wait()
    def iw_fire(s, _):
        pltpu.async_copy(
            ibuf.at[s],
            iflat_hbm.at[pl.ds((s * NBT + w) * 128, 128)], sem_iw)
        return 0

    lax.fori_loop(0, SEQ, iw_fire, 0)

    def plane_fire(c, p):
        @plsc.parallel_loop(0, EMBED, step=1, unroll=4)
        def _fire(d):
            pltpu.async_copy(
                inbuf.at[p, d],
                tpl_hbm.at[pl.ds(d * PLANE_STRIDE + c * 128, 128)], sem_pw)

    def plane_drain(c, p):
        pltpu.make_async_copy(
            tpl_hbm.at[pl.ds(0, 8192)], dummy, sem_pw).wait()

    def in_copy(i, p):
        c = i * NW + w
        return pltpu.async_copy(
            table_hbm.at[:, pl.ds(pl.multiple_of(c * 128, 128), 128)],
            inbuf.at[p], sem_in)

    NCOL = 24
    loads = {0: in_copy(0, 0)}
    for i in range(NCOL):
        p = i % 2
        loads[i].wait()
        if i >= 1:
            plane_drain((i - 1) * NW + w, (i - 1) % 2)
        if i + 1 < NCOL:
            loads[i + 1] = in_copy(i + 1, (i + 1) % 2)
        plane_fire(i * NW + w, p)
    plane_drain((NCOL - 1) * NW + w, (NCOL - 1) % 2)

    @pl.when(w < NTILECOL - NCOL * NW)
    def _tail():
        c = NCOL * NW + w
        pltpu.async_copy(
            table_hbm.at[:, pl.ds(pl.multiple_of(c * 128, 128), 128)],
            inbuf.at[0], sem_in).wait()
        plane_fire(c, 0)
        plane_drain(c, 0)

    pltpu.make_async_copy(
        tpl_hbm.at[pl.ds(0, SEQ * 128)], dummy.at[pl.ds(0, SEQ * 128)],
        sem_iw).wait()


def _gather_body(iflat_hbm, tpl_hbm, out_hbm,
                 idxrow, planebuf, sbuf, sem_p, sem_i, sem_s):
    w = lax.axis_index("s") * 2 + lax.axis_index("c")

    def do_plane(r, _):
        d = w + r * 32
        dt = lax.shift_right_logical(d, 3)
        d8 = lax.bitwise_and(d, 7)
        pltpu.async_copy(
            tpl_hbm.at[pl.ds(pl.multiple_of(d * PLANE_STRIDE, 8), VOCAB)],
            planebuf, sem_p).wait()

        def idx_load(s):
            return pltpu.async_copy(
                iflat_hbm.at[pl.ds(s * BATCH, BATCH)], idxrow.at[s % 2],
                sem_i)

        iloads = {0: idx_load(0), 1: idx_load(1)}
        stores = {}
        for s in range(SEQ):
            p = s % 2
            p3 = s % 3
            iloads[s].wait()
            if s + 2 < SEQ:
                iloads[s + 2] = idx_load(s + 2)
            if s >= 3:
                stores[s - 3].wait()

            @plsc.parallel_loop(0, BATCH, step=16, unroll=16)
            def _gather(j):
                ivals = idxrow[p, pl.ds(j, 16)]
                vals = plsc.load_gather(planebuf, [ivals])
                bt = lax.shift_right_logical(j, 7)
                off = lax.bitwise_and(j, 127)
                sbuf[p3, bt, pl.ds(off, 16)] = vals

            stores[s] = pltpu.async_copy(
                sbuf.at[p3], out_hbm.at[s, dt, :, d8, :], sem_s)
        for s in range(SEQ - 3, SEQ):
            stores[s].wait()
        return 0

    lax.fori_loop(0, 2, do_plane, 0)


@jax.jit
def _lookup(idx_t, table_t):
    mesh = plsc.VectorSubcoreMesh(core_axis_name="c", subcore_axis_name="s")
    tpl, iflat = pl.kernel(
        _detile_body,
        out_type=(
            jax.ShapeDtypeStruct((FLAT_TABLE,), jnp.float32),
            jax.ShapeDtypeStruct((FLAT_IDX,), jnp.int32),
        ),
        mesh=mesh,
        scratch_types=[
            pltpu.VMEM((2, EMBED, 128), jnp.float32),
            pltpu.VMEM((56, 128), jnp.int32),
            pltpu.VMEM((8192,), jnp.float32),
            pltpu.SemaphoreType.DMA,
            pltpu.SemaphoreType.DMA,
            pltpu.SemaphoreType.DMA,
            pltpu.SemaphoreType.DMA,
        ],
        compiler_params=pltpu.CompilerParams(
            use_tc_tiling_on_sc=True, needs_layout_passes=False),
    )(idx_t, table_t)

    mesh2 = plsc.VectorSubcoreMesh(core_axis_name="c", subcore_axis_name="s")
    out5 = pl.kernel(
        _gather_body,
        out_type=jax.ShapeDtypeStruct((SEQ, 8, NBT, 8, 128), jnp.float32),
        mesh=mesh2,
        scratch_types=[
            pltpu.VMEM((2, BATCH), jnp.int32),
            pltpu.VMEM((VOCAB,), jnp.float32),
            pltpu.VMEM((3, NBT, 128), jnp.float32),
            pltpu.SemaphoreType.DMA,
            pltpu.SemaphoreType.DMA,
            pltpu.SemaphoreType.DMA,
        ],
        compiler_params=pltpu.CompilerParams(
            use_tc_tiling_on_sc=False, needs_layout_passes=False),
    )(iflat, tpl)
    return out5


def kernel(token_type_ids, table):
    idx_t = jnp.transpose(token_type_ids, (1, 0))
    table_t = jnp.transpose(table, (1, 0))
    out5 = _lookup(idx_t, table_t)
    out = jnp.transpose(out5, (2, 4, 0, 1, 3))
    return jnp.reshape(out, (BATCH, SEQ, EMBED))


# speedup vs baseline: 1.5087x; 1.3437x over previous
import jax
import jax.numpy as jnp
from jax import lax
from jax.experimental import pallas as pl
from jax.experimental.pallas import tpu as pltpu
from jax.experimental.pallas import tpu_sc as plsc

VOCAB = 100000
PLANE_STRIDE = 100096
EMBED = 64
BATCH = 4096
SEQ = 50
NBT = BATCH // 128
NW = 32
NTILECOL = PLANE_STRIDE // 128

FLAT_TABLE = EMBED * PLANE_STRIDE
FLAT_IDX = SEQ * BATCH


def _detile_body(idx_hbm, table_hbm, tpl_hbm, iflat_hbm,
                 inbuf, ibuf, dummy, sem_in, sem_pw, sem_ib, sem_iw):
    w = lax.axis_index("s") * 2 + lax.axis_index("c")

    ilds = []
    for st in range(7):
        rows = 8 if st < 6 else 2
        ilds.append(pltpu.async_copy(
            idx_hbm.at[pl.ds(st * 8, rows),
                       pl.ds(pl.multiple_of(w * 128, 128), 128)],
            ibuf.at[pl.ds(st * 8, rows)], sem_ib))
    for c in ilds:
        c.wait()
    def iw_fire(s, _):
        pltpu.async_copy(
            ibuf.at[s],
            iflat_hbm.at[pl.ds((s * NBT + w) * 128, 128)], sem_iw)
        return 0

    lax.fori_loop(0, SEQ, iw_fire, 0)

    def plane_fire(c, p):
        @plsc.parallel_loop(0, EMBED, step=1, unroll=4)
        def _fire(d):
            pltpu.async_copy(
                inbuf.at[p, d],
                tpl_hbm.at[pl.ds(d * PLANE_STRIDE + c * 128, 128)], sem_pw)

    def plane_drain(c, p):
        pltpu.make_async_copy(
            tpl_hbm.at[pl.ds(0, 8192)], dummy, sem_pw).wait()

    def in_copy(i, p):
        c = i * NW + w
        return pltpu.async_copy(
            table_hbm.at[:, pl.ds(pl.multiple_of(c * 128, 128), 128)],
            inbuf.at[p], sem_in)

    NCOL = 24
    loads = {0: in_copy(0, 0)}
    for i in range(NCOL):
        p = i % 2
        loads[i].wait()
        if i >= 1:
            plane_drain((i - 1) * NW + w, (i - 1) % 2)
        if i + 1 < NCOL:
            loads[i + 1] = in_copy(i + 1, (i + 1) % 2)
        plane_fire(i * NW + w, p)
    plane_drain((NCOL - 1) * NW + w, (NCOL - 1) % 2)

    @pl.when(w < NTILECOL - NCOL * NW)
    def _tail():
        c = NCOL * NW + w
        pltpu.async_copy(
            table_hbm.at[:, pl.ds(pl.multiple_of(c * 128, 128), 128)],
            inbuf.at[0], sem_in).wait()
        plane_fire(c, 0)
        plane_drain(c, 0)

    pltpu.make_async_copy(
        tpl_hbm.at[pl.ds(0, SEQ * 128)], dummy.at[pl.ds(0, SEQ * 128)],
        sem_iw).wait()


def _gather_body(iflat_hbm, tpl_hbm, out_hbm,
                 idxrow, planebuf, sbuf, spidx, sem_p, sem_i, sem_s, sem_st):
    sid = lax.axis_index("s")
    w = sid * 2 + lax.axis_index("c")

    pltpu.async_copy(
        iflat_hbm.at[pl.ds(sid * (FLAT_IDX // 16), FLAT_IDX // 16)],
        spidx.at[pl.ds(sid * (FLAT_IDX // 16), FLAT_IDX // 16)],
        sem_st).wait()
    plsc.subcore_barrier()

    def do_plane(r, _):
        d = w + r * 32
        dt = lax.shift_right_logical(d, 3)
        d8 = lax.bitwise_and(d, 7)
        pltpu.async_copy(
            tpl_hbm.at[pl.ds(pl.multiple_of(d * PLANE_STRIDE, 8), VOCAB)],
            planebuf, sem_p).wait()

        def idx_load(s):
            return pltpu.async_copy(
                spidx.at[pl.ds(s * BATCH, BATCH)], idxrow.at[s % 2],
                sem_i)

        iloads = {0: idx_load(0), 1: idx_load(1)}
        stores = {}
        for s in range(SEQ):
            p = s % 2
            p3 = s % 2
            iloads[s].wait()
            if s + 2 < SEQ:
                iloads[s + 2] = idx_load(s + 2)
            if s >= 2:
                stores[s - 2].wait()

            @plsc.parallel_loop(0, BATCH, step=16, unroll=16)
            def _gather(j):
                ivals = idxrow[p, pl.ds(j, 16)]
                vals = plsc.load_gather(planebuf, [ivals])
                bt = lax.shift_right_logical(j, 7)
                off = lax.bitwise_and(j, 127)
                sbuf[p3, bt, pl.ds(off, 16)] = vals

            stores[s] = pltpu.async_copy(
                sbuf.at[p3], out_hbm.at[s, dt, :, d8, :], sem_s)
        for s in range(SEQ - 2, SEQ):
            stores[s].wait()
        return 0

    lax.fori_loop(0, 2, do_plane, 0)


@jax.jit
def _lookup(idx_t, table_t):
    mesh = plsc.VectorSubcoreMesh(core_axis_name="c", subcore_axis_name="s")
    tpl, iflat = pl.kernel(
        _detile_body,
        out_type=(
            jax.ShapeDtypeStruct((FLAT_TABLE,), jnp.float32),
            jax.ShapeDtypeStruct((FLAT_IDX,), jnp.int32),
        ),
        mesh=mesh,
        scratch_types=[
            pltpu.VMEM((2, EMBED, 128), jnp.float32),
            pltpu.VMEM((56, 128), jnp.int32),
            pltpu.VMEM((8192,), jnp.float32),
            pltpu.SemaphoreType.DMA,
            pltpu.SemaphoreType.DMA,
            pltpu.SemaphoreType.DMA,
            pltpu.SemaphoreType.DMA,
        ],
        compiler_params=pltpu.CompilerParams(
            use_tc_tiling_on_sc=True, needs_layout_passes=False),
    )(idx_t, table_t)

    mesh2 = plsc.VectorSubcoreMesh(core_axis_name="c", subcore_axis_name="s")
    out5 = pl.kernel(
        _gather_body,
        out_type=jax.ShapeDtypeStruct((SEQ, 8, NBT, 8, 128), jnp.float32),
        mesh=mesh2,
        scratch_types=[
            pltpu.VMEM((2, BATCH), jnp.int32),
            pltpu.VMEM((VOCAB,), jnp.float32),
            pltpu.VMEM((2, NBT, 128), jnp.float32),
            pltpu.VMEM_SHARED((FLAT_IDX,), jnp.int32),
            pltpu.SemaphoreType.DMA,
            pltpu.SemaphoreType.DMA,
            pltpu.SemaphoreType.DMA,
            pltpu.SemaphoreType.DMA,
        ],
        compiler_params=pltpu.CompilerParams(
            use_tc_tiling_on_sc=False, needs_layout_passes=False),
    )(iflat, tpl)
    return out5


def kernel(token_type_ids, table):
    idx_t = jnp.transpose(token_type_ids, (1, 0))
    table_t = jnp.transpose(table, (1, 0))
    out5 = _lookup(idx_t, table_t)
    out = jnp.transpose(out5, (2, 4, 0, 1, 3))
    return jnp.reshape(out, (BATCH, SEQ, EMBED))
